# baseline (speedup 1.0000x reference)
import jax
import jax.numpy as jnp
from jax import lax
from jax.experimental import pallas as pl
from jax.experimental.pallas import tpu as pltpu
from jax.experimental.pallas import tpu_sc as plsc

N = 10000
E = 160000
F = 256
ALPHA = 0.2

NC = 2
NS = 16
NT = NC * NS
BW = 320
TRASH = BW
ACC_R = 328
G = 128
SLOTS = E + NT * G

BE = 256
NBLK = E // BE

PT = 5120
BULK = 1280

f32 = jnp.float32
i32 = jnp.int32


def _hist_body(d3, h3):
    d = d3[...].reshape(BE, 1)
    b = d // BW
    oh = (b == lax.broadcasted_iota(i32, (BE, 128), 1)).astype(f32)
    h3[...] = jnp.sum(oh, axis=0).reshape(1, 1, 128)


def _pref_body(h3, sb3, meta):
    h = h3[...].reshape(NBLK, 128)
    rl = lax.broadcasted_iota(i32, (NBLK, NBLK), 0)
    cl = lax.broadcasted_iota(i32, (NBLK, NBLK), 1)
    strict_l = (cl < rl).astype(f32)
    bo = jnp.dot(strict_l, h, preferred_element_type=f32)
    tot = jnp.sum(h, axis=0, keepdims=True)
    caps = jnp.floor((tot + (G - 1)) * (1.0 / G)) * G
    ru = lax.broadcasted_iota(i32, (128, 128), 0)
    cu = lax.broadcasted_iota(i32, (128, 128), 1)
    strict_u = (ru < cu).astype(f32)
    starts = jnp.dot(caps, strict_u, preferred_element_type=f32)
    sb3[...] = (starts + bo).reshape(NBLK, 1, 128)
    meta[...] = jnp.concatenate(
        [starts, tot, caps * (1.0 / G), jnp.zeros((5, 128), f32)],
        axis=0).astype(i32)


def _pos_body(d3, s3, sb3, pos3, pk3):
    d = d3[...].reshape(BE, 1)
    src = s3[...].reshape(BE, 1)
    b = d // BW
    oh = (b == lax.broadcasted_iota(i32, (BE, 128), 1)).astype(f32)
    rl = lax.broadcasted_iota(i32, (BE, BE), 0)
    cl = lax.broadcasted_iota(i32, (BE, BE), 1)
    strict_l = (cl < rl).astype(f32)
    wr = jnp.dot(strict_l, oh, preferred_element_type=f32)
    sb = sb3[...].reshape(1, 128)
    posf = jnp.sum(oh * (wr + sb), axis=1, keepdims=True)
    pos3[...] = posf.astype(i32).reshape(1, BE, 1)
    pk3[...] = (src + (d - b * BW) * 16384).reshape(1, BE, 1)


_d3_spec = pl.BlockSpec((1, BE, 1), lambda i: (i, 0, 0))
_h3_spec = pl.BlockSpec((1, 1, 128), lambda i: (i, 0, 0))

_hist = pl.pallas_call(
    _hist_body,
    grid=(NBLK,),
    in_specs=[_d3_spec],
    out_specs=_h3_spec,
    out_shape=jax.ShapeDtypeStruct((NBLK, 1, 128), f32),
)

_pref = pl.pallas_call(
    _pref_body,
    grid=(1,),
    in_specs=[pl.BlockSpec((NBLK, 1, 128), lambda i: (0, 0, 0))],
    out_specs=[pl.BlockSpec((NBLK, 1, 128), lambda i: (0, 0, 0)),
               pl.BlockSpec((8, 128), lambda i: (0, 0))],
    out_shape=[jax.ShapeDtypeStruct((NBLK, 1, 128), f32),
               jax.ShapeDtypeStruct((8, 128), i32)],
)

_pos = pl.pallas_call(
    _pos_body,
    grid=(NBLK,),
    in_specs=[_d3_spec, _d3_spec, _h3_spec],
    out_specs=[_d3_spec, _d3_spec],
    out_shape=[jax.ShapeDtypeStruct((NBLK, BE, 1), i32),
               jax.ShapeDtypeStruct((NBLK, BE, 1), i32)],
)


def _perm_one(pos_hbm, pk_hbm, pks_hbm, posv, pkv, idx, sem, wid):
    base = wid * PT
    nb = jnp.where(wid < NT - 1, PT // BULK, 1)

    def _bulk(j, carry):
        pltpu.sync_copy(pos_hbm.at[pl.ds(base + j * BULK, BULK)],
                        posv.at[pl.ds(j * BULK, BULK)])
        pltpu.sync_copy(pk_hbm.at[pl.ds(base + j * BULK, BULK)],
                        pkv.at[pl.ds(j * BULK, BULK)])
        return carry

    lax.fori_loop(0, nb, _bulk, 0)
    nchk = jnp.where(wid < NT - 1, PT // G, BULK // G)
    depth = 4

    def _drain():
        pltpu.make_async_copy(pkv.at[pl.ds(0, G)], pks_hbm.at[idx.at[0]],
                              sem).wait()

    def _chunk(j, carry):
        @pl.when(j >= depth)
        def _dr():
            _drain()

        pg = j - (j // depth) * depth
        for k in range(G // 16):
            idx[pg, pl.ds(k * 16, 16)] = posv[pl.ds(j * G + k * 16, 16)]
        pltpu.async_copy(pkv.at[pl.ds(j * G, G)], pks_hbm.at[idx.at[pg]], sem)
        return carry

    lax.fori_loop(0, nchk, _chunk, 0)
    ndr = jnp.minimum(nchk, depth)

    def _fin(j, carry):
        _drain()
        return carry

    lax.fori_loop(0, ndr, _fin, 0)


def _perm_body(pos_a, pk_a, pos_b, pk_b, pks_a, pks_b, posv, pkv, idx, sem):
    c = lax.axis_index("c")
    s = lax.axis_index("s")
    wid = s * NC + c
    _perm_one(pos_a, pk_a, pks_a, posv, pkv, idx, sem, wid)
    _perm_one(pos_b, pk_b, pks_b, posv, pkv, idx, sem, wid)


import functools


@functools.cache
def _sc_mesh():
    return plsc.VectorSubcoreMesh(core_axis_name="c", subcore_axis_name="s",
                                  num_cores=NC, num_subcores=NS)


@functools.cache
def _perm_kernel():
    return pl.kernel(
        _perm_body,
        out_type=(jax.ShapeDtypeStruct((SLOTS,), i32),
                  jax.ShapeDtypeStruct((SLOTS,), i32)),
        mesh=_sc_mesh(),
        scratch_types=[
            pltpu.VMEM((PT,), i32),
            pltpu.VMEM((PT,), i32),
            pltpu.VMEM((4, G), i32),
            pltpu.SemaphoreType.DMA,
        ],
        name="sc_permute",
    )


def _lane(v, r):
    out = jnp.int32(0)
    for j in range(16):
        out = jnp.where(r == j, v[j], out)
    return out


GS = 64


def _spmm_one(x_hbm, pks_hbm, meta_hbm, out_hbm,
              metav, pkbuf, gidx, lbuf, rows, accs, semi, semg, wid):
    jv = wid // 16
    r = wid - jv * 16
    pltpu.sync_copy(meta_hbm.at[pl.ds(0, 384)], metav)
    start_t = pl.multiple_of(_lane(metav[pl.ds(jv * 16, 16)], r), G)
    cnt_t = _lane(metav[pl.ds(128 + jv * 16, 16)], r)
    nch = _lane(metav[pl.ds(256 + jv * 16, 16)], r) * (G // GS)

    z = jnp.zeros((16,), f32)

    def _zero(i, carry):
        for k in range(F // 16):
            accs[k][pl.ds(i * 16, 16)] = z
        return carry

    lax.fori_loop(0, ACC_R, _zero, 0)

    def _par(j):
        return j - (j // 2) * 2

    def _fire_idx(j):
        pg = _par(j)
        bs = pl.multiple_of(start_t + j * GS, GS)
        pltpu.async_copy(pks_hbm.at[pl.ds(bs, GS)],
                         pkbuf.at[pl.ds(pg * GS, GS)], semi)

    def _wait_idx():
        pltpu.make_async_copy(pks_hbm.at[pl.ds(0, GS)],
                              pkbuf.at[pl.ds(0, GS)], semi).wait()

    def _unpack(j):
        pg = _par(j)
        for k in range(GS // 16):
            li = lax.iota(i32, 16) + (j * GS + k * 16)
            m = li < cnt_t
            o = pg * GS + k * 16
            pk = pkbuf[pl.ds(o, 16)]
            gidx[pl.ds(o, 16)] = jnp.where(m, pk & 16383, 0)
            lbuf[pl.ds(o, 16)] = jnp.where(m, pk >> 14, TRASH)

    def _fire_gather(j):
        pg = _par(j)
        pltpu.async_copy(x_hbm.at[gidx.at[pl.ds(pg * GS, GS)]],
                         rows.at[pl.ds(pg * GS, GS)], semg)

    def _wait_gather():
        pltpu.make_async_copy(x_hbm.at[gidx.at[pl.ds(0, GS)]],
                              rows.at[pl.ds(0, GS)], semg).wait()

    @pl.when(nch > 0)
    def _prologue():
        _fire_idx(0)
        _wait_idx()
        _unpack(0)
        _fire_gather(0)

        @pl.when(nch > 1)
        def _p2():
            _fire_idx(1)

    def _chunk(j, carry):
        @pl.when(j + 1 < nch)
        def _nxt():
            _wait_idx()
            _unpack(j + 1)
            _fire_gather(j + 1)

        _wait_gather()
        pg = _par(j)

        def _grp(g, c2):
            lvec = lbuf[pl.ds(pg * GS + g * 16, 16)]
            abs_ = [pl.multiple_of(lvec[jj] * 16, 16) for jj in range(16)]
            for jj in range(16):
                ab = abs_[jj]
                e = pg * GS + g * 16 + jj
                for k in range(F // 16):
                    accs[k][pl.ds(ab, 16)] = (
                        accs[k][pl.ds(ab, 16)]
                        + rows[e, pl.ds(k * 16, 16)])
            return c2

        lax.fori_loop(0, GS // 16, _grp, 0)

        @pl.when(j + 2 < nch)
        def _pref():
            _fire_idx(j + 2)

        return carry

    lax.fori_loop(0, nch, _chunk, 0)

    outsz = BW * 16
    lastsz = (N - (NT - 1) * BW) * 16

    for k in range(F // 16):
        @pl.when(wid < NT - 1)
        def _co(k=k):
            pltpu.sync_copy(accs[k].at[pl.ds(0, outsz)],
                            out_hbm.at[k, pl.ds(wid * outsz, outsz)])

        @pl.when(wid == NT - 1)
        def _co_last(k=k):
            pltpu.sync_copy(accs[k].at[pl.ds(0, lastsz)],
                            out_hbm.at[k, pl.ds((NT - 1) * outsz, lastsz)])


def _spmm_body(xa_hbm, pks_a, meta_a, xb_hbm, pks_b, meta_b,
               out_a, out_b,
               metav, pkbuf, gidx, lbuf, rows,
               a0, a1, a2, a3, a4, a5, a6, a7,
               a8, a9, a10, a11, a12, a13, a14, a15, semi, semg):
    accs = (a0, a1, a2, a3, a4, a5, a6, a7,
            a8, a9, a10, a11, a12, a13, a14, a15)
    c = lax.axis_index("c")
    s = lax.axis_index("s")
    wid = s * NC + c
    _spmm_one(xa_hbm, pks_a, meta_a, out_a,
              metav, pkbuf, gidx, lbuf, rows, accs, semi, semg, wid)
    _spmm_one(xb_hbm, pks_b, meta_b, out_b,
              metav, pkbuf, gidx, lbuf, rows, accs, semi, semg, wid)


@functools.cache
def _spmm_kernel():
    return pl.kernel(
        _spmm_body,
        out_type=(jax.ShapeDtypeStruct((F // 16, N * 16), f32),
                  jax.ShapeDtypeStruct((F // 16, N * 16), f32)),
        mesh=_sc_mesh(),
        scratch_types=[
            pltpu.VMEM((384,), i32),
            pltpu.VMEM((2 * GS,), i32),
            pltpu.VMEM((2 * GS,), i32),
            pltpu.VMEM((2 * GS,), i32),
            pltpu.VMEM((2 * GS, F), f32),
        ] + [pltpu.VMEM((ACC_R * 16,), f32) for _ in range(F // 16)] + [
            pltpu.SemaphoreType.DMA,
            pltpu.SemaphoreType.DMA,
        ],
        name="sc_spmm",
    )


BLK = 1000


def _leaky(x):
    return jnp.where(x >= 0, x, ALPHA * x)


def _k1_body(u, v, w1, w2, wu2, wi2, bu, bi, s1, s2, pu, pv):
    uf = u[...]
    vf = v[...]
    s1[...] = jnp.dot(uf, w1[...], preferred_element_type=f32)
    s2[...] = jnp.dot(vf, w2[...], preferred_element_type=f32)
    pu[...] = jnp.dot(uf, wu2[...], preferred_element_type=f32) + bu[...]
    pv[...] = jnp.dot(vf, wi2[...], preferred_element_type=f32) + bi[...]


def _k3_body(t1, b1, w3, t2, b2, w4, s3, s4):
    h1 = _leaky(t1[...] + b1[...])
    h2 = _leaky(t2[...] + b2[...])
    s3[...] = jnp.dot(h1, w3[...], preferred_element_type=f32)
    s4[...] = jnp.dot(h2, w4[...], preferred_element_type=f32)


def _k5_body(t3, b3, wu1, pu, t4, b4, wi1, pv, user, item):
    h3 = _leaky(t3[...] + b3[...])
    h4 = _leaky(t4[...] + b4[...])
    user[...] = jnp.dot(h3, wu1[...], preferred_element_type=f32) + pu[...]
    item[...] = jnp.dot(h4, wi1[...], preferred_element_type=f32) + pv[...]


_x_spec = pl.BlockSpec((BLK, F), lambda i: (i, 0))
_w_spec = pl.BlockSpec((F, F), lambda i: (0, 0))
_b_spec = pl.BlockSpec((1, F), lambda i: (0, 0))
_o_sd = jax.ShapeDtypeStruct((N, F), f32)

_k1 = pl.pallas_call(
    _k1_body,
    grid=(N // BLK,),
    in_specs=[_x_spec, _x_spec, _w_spec, _w_spec, _w_spec, _w_spec,
              _b_spec, _b_spec],
    out_specs=[_x_spec, _x_spec, _x_spec, _x_spec],
    out_shape=[_o_sd, _o_sd, _o_sd, _o_sd],
)

_k3 = pl.pallas_call(
    _k3_body,
    grid=(N // BLK,),
    in_specs=[_x_spec, _b_spec, _w_spec, _x_spec, _b_spec, _w_spec],
    out_specs=[_x_spec, _x_spec],
    out_shape=[_o_sd, _o_sd],
)

_k5 = pl.pallas_call(
    _k5_body,
    grid=(N // BLK,),
    in_specs=[_x_spec, _b_spec, _w_spec, _x_spec,
              _x_spec, _b_spec, _w_spec, _x_spec],
    out_specs=[_x_spec, _x_spec],
    out_shape=[_o_sd, _o_sd],
)


def _rank(dst, src):
    d3 = dst.reshape(NBLK, BE, 1)
    h3 = _hist(d3)
    sb3, meta8 = _pref(h3)
    pos3, pk3 = _pos(d3, src.reshape(NBLK, BE, 1), sb3)
    return pos3.reshape(E), pk3.reshape(E), meta8.reshape(1024)


def kernel(ufea, vfea, UV_adj, VU_adj, gc1_W, gc1_b, gc2_W, gc2_b,
           gc3m_W, gc3m_b, gc3s_W, gc3s_b, gc4m_W, gc4m_b, gc4s_W, gc4s_b,
           uum_W, uum_b, uus_W, uus_b, ium_W, ium_b, ius_W, ius_b):
    u_idx, i_idx = UV_adj[0], UV_adj[1]
    pos_u, pk_u, meta_u = _rank(u_idx, i_idx)
    pos_i, pk_i, meta_i = _rank(i_idx, u_idx)
    pks_u, pks_i = _perm_kernel()(pos_u, pk_u, pos_i, pk_i)

    def _unchunk(o):
        return o.reshape(F // 16, N, 16).transpose(1, 0, 2).reshape(N, F)

    s1, s2, pu, pv = _k1(ufea, vfea, gc1_W, gc2_W, uum_W[F:], ium_W[F:],
                         uum_b.reshape(1, F), ium_b.reshape(1, F))
    spmm = _spmm_kernel()
    t1c, t2c = spmm(s1, pks_i, meta_i, s2, pks_u, meta_u)
    t1, t2 = _unchunk(t1c), _unchunk(t2c)
    s3, s4 = _k3(t1, gc1_b.reshape(1, F), gc3m_W,
                 t2, gc2_b.reshape(1, F), gc4m_W)
    t3c, t4c = spmm(s3, pks_u, meta_u, s4, pks_i, meta_i)
    t3, t4 = _unchunk(t3c), _unchunk(t4c)
    user, item = _k5(t3, gc3m_b.reshape(1, F), uum_W[:F], pu,
                     t4, gc4m_b.reshape(1, F), ium_W[:F], pv)
    return (user, item)


# speedup vs baseline: 1.4263x; 1.4263x over previous
import jax
import jax.numpy as jnp
from jax import lax
from jax.experimental import pallas as pl
from jax.experimental.pallas import tpu as pltpu
from jax.experimental.pallas import tpu_sc as plsc

N = 10000
E = 160000
F = 256
ALPHA = 0.2

NC = 2
NS = 16
NT = NC * NS
BW = 320
TRASH = BW
ACC_R = 328
G = 128
SLOTS = E + NT * G

BE = 256
NBLK = E // BE

PT = 5120
BULK = 1280

f32 = jnp.float32
i32 = jnp.int32


SB = 25


def _hist_body(d3, h3):
    for b_i in range(SB):
        d = d3[b_i].reshape(BE, 1)
        b = d // BW
        oh = (b == lax.broadcasted_iota(i32, (BE, 128), 1)).astype(f32)
        h3[b_i] = jnp.sum(oh, axis=0).reshape(1, 128)


def _pref_body(h3, sb3, meta):
    h = h3[...].reshape(NBLK, 128)
    rl = lax.broadcasted_iota(i32, (NBLK, NBLK), 0)
    cl = lax.broadcasted_iota(i32, (NBLK, NBLK), 1)
    strict_l = (cl < rl).astype(f32)
    bo = jnp.dot(strict_l, h, preferred_element_type=f32)
    tot = jnp.sum(h, axis=0, keepdims=True)
    caps = jnp.floor((tot + (G - 1)) * (1.0 / G)) * G
    ru = lax.broadcasted_iota(i32, (128, 128), 0)
    cu = lax.broadcasted_iota(i32, (128, 128), 1)
    strict_u = (ru < cu).astype(f32)
    starts = jnp.dot(caps, strict_u, preferred_element_type=f32)
    sb3[...] = (starts + bo).reshape(NBLK, 1, 128)
    meta[...] = jnp.concatenate(
        [starts, tot, caps * (1.0 / G), jnp.zeros((5, 128), f32)],
        axis=0).astype(i32)


def _pos_body(d3, s3, sb3, pos3, pk3):
    rl = lax.broadcasted_iota(i32, (BE, BE), 0)
    cl = lax.broadcasted_iota(i32, (BE, BE), 1)
    strict_l = (cl < rl).astype(f32)
    for b_i in range(SB):
        d = d3[b_i].reshape(BE, 1)
        src = s3[b_i].reshape(BE, 1)
        b = d // BW
        oh = (b == lax.broadcasted_iota(i32, (BE, 128), 1)).astype(f32)
        wr = jnp.dot(strict_l, oh, preferred_element_type=f32)
        sb = sb3[b_i].reshape(1, 128)
        posf = jnp.sum(oh * (wr + sb), axis=1, keepdims=True)
        pos3[b_i] = posf.astype(i32).reshape(BE, 1)
        pk3[b_i] = src + (d - b * BW) * 16384


_d3_spec = pl.BlockSpec((SB, BE, 1), lambda i: (i, 0, 0))
_h3_spec = pl.BlockSpec((SB, 1, 128), lambda i: (i, 0, 0))

_hist = pl.pallas_call(
    _hist_body,
    grid=(NBLK // SB,),
    in_specs=[_d3_spec],
    out_specs=_h3_spec,
    out_shape=jax.ShapeDtypeStruct((NBLK, 1, 128), f32),
)

_pref = pl.pallas_call(
    _pref_body,
    grid=(1,),
    in_specs=[pl.BlockSpec((NBLK, 1, 128), lambda i: (0, 0, 0))],
    out_specs=[pl.BlockSpec((NBLK, 1, 128), lambda i: (0, 0, 0)),
               pl.BlockSpec((8, 128), lambda i: (0, 0))],
    out_shape=[jax.ShapeDtypeStruct((NBLK, 1, 128), f32),
               jax.ShapeDtypeStruct((8, 128), i32)],
)

_pos = pl.pallas_call(
    _pos_body,
    grid=(NBLK // SB,),
    in_specs=[_d3_spec, _d3_spec, _h3_spec],
    out_specs=[_d3_spec, _d3_spec],
    out_shape=[jax.ShapeDtypeStruct((NBLK, BE, 1), i32),
               jax.ShapeDtypeStruct((NBLK, BE, 1), i32)],
)


def _perm_one(pos_hbm, pk_hbm, pks_hbm, posv, pkv, idx, sem, wid):
    base = wid * PT
    nb = jnp.where(wid < NT - 1, PT // BULK, 1)

    def _bulk(j, carry):
        pltpu.sync_copy(pos_hbm.at[pl.ds(base + j * BULK, BULK)],
                        posv.at[pl.ds(j * BULK, BULK)])
        pltpu.sync_copy(pk_hbm.at[pl.ds(base + j * BULK, BULK)],
                        pkv.at[pl.ds(j * BULK, BULK)])
        return carry

    lax.fori_loop(0, nb, _bulk, 0)
    nchk = jnp.where(wid < NT - 1, PT // G, BULK // G)
    depth = 4

    def _drain():
        pltpu.make_async_copy(pkv.at[pl.ds(0, G)], pks_hbm.at[idx.at[0]],
                              sem).wait()

    def _chunk(j, carry):
        @pl.when(j >= depth)
        def _dr():
            _drain()

        pg = j - (j // depth) * depth
        for k in range(G // 16):
            idx[pg, pl.ds(k * 16, 16)] = posv[pl.ds(j * G + k * 16, 16)]
        pltpu.async_copy(pkv.at[pl.ds(j * G, G)], pks_hbm.at[idx.at[pg]], sem)
        return carry

    lax.fori_loop(0, nchk, _chunk, 0)
    ndr = jnp.minimum(nchk, depth)

    def _fin(j, carry):
        _drain()
        return carry

    lax.fori_loop(0, ndr, _fin, 0)


def _perm_body(pos_a, pk_a, pos_b, pk_b, pks_a, pks_b, posv, pkv, idx, sem):
    c = lax.axis_index("c")
    s = lax.axis_index("s")
    wid = s * NC + c
    _perm_one(pos_a, pk_a, pks_a, posv, pkv, idx, sem, wid)
    _perm_one(pos_b, pk_b, pks_b, posv, pkv, idx, sem, wid)


import functools


@functools.cache
def _sc_mesh():
    return plsc.VectorSubcoreMesh(core_axis_name="c", subcore_axis_name="s",
                                  num_cores=NC, num_subcores=NS)


@functools.cache
def _perm_kernel():
    return pl.kernel(
        _perm_body,
        out_type=(jax.ShapeDtypeStruct((SLOTS,), i32),
                  jax.ShapeDtypeStruct((SLOTS,), i32)),
        mesh=_sc_mesh(),
        scratch_types=[
            pltpu.VMEM((PT,), i32),
            pltpu.VMEM((PT,), i32),
            pltpu.VMEM((4, G), i32),
            pltpu.SemaphoreType.DMA,
        ],
        name="sc_permute",
    )


def _lane(v, r):
    out = jnp.int32(0)
    for j in range(16):
        out = jnp.where(r == j, v[j], out)
    return out


GS = 64


def _spmm_one(x_hbm, pks_hbm, meta_hbm, out_hbm,
              metav, pkbuf, gidx, lbuf, rows, acc, semi, semg, wid):
    jv = wid // 16
    r = wid - jv * 16
    pltpu.sync_copy(meta_hbm.at[pl.ds(0, 384)], metav)
    start_t = pl.multiple_of(_lane(metav[pl.ds(jv * 16, 16)], r), G)
    cnt_t = _lane(metav[pl.ds(128 + jv * 16, 16)], r)
    nch = _lane(metav[pl.ds(256 + jv * 16, 16)], r) * (G // GS)

    z = jnp.zeros((16,), f32)

    def _zero(i, carry):
        for k in range(F // 16):
            acc[pl.ds(i * F + k * 16, 16)] = z
        return carry

    lax.fori_loop(0, ACC_R, _zero, 0)

    def _par(j):
        return j - (j // 2) * 2

    def _fire_idx(j):
        pg = _par(j)
        bs = pl.multiple_of(start_t + j * GS, GS)
        pltpu.async_copy(pks_hbm.at[pl.ds(bs, GS)],
                         pkbuf.at[pl.ds(pg * GS, GS)], semi)

    def _wait_idx():
        pltpu.make_async_copy(pks_hbm.at[pl.ds(0, GS)],
                              pkbuf.at[pl.ds(0, GS)], semi).wait()

    def _unpack(j):
        pg = _par(j)
        for k in range(GS // 16):
            li = lax.iota(i32, 16) + (j * GS + k * 16)
            m = li < cnt_t
            o = pg * GS + k * 16
            pk = pkbuf[pl.ds(o, 16)]
            gidx[pl.ds(o, 16)] = jnp.where(m, pk & 16383, 0)
            lbuf[pl.ds(o, 16)] = jnp.where(m, pk >> 14, TRASH)

    def _fire_gather(j):
        pg = _par(j)
        pltpu.async_copy(x_hbm.at[gidx.at[pl.ds(pg * GS, GS)]],
                         rows.at[pl.ds(pg * GS, GS)], semg)

    def _wait_gather():
        pltpu.make_async_copy(x_hbm.at[gidx.at[pl.ds(0, GS)]],
                              rows.at[pl.ds(0, GS)], semg).wait()

    @pl.when(nch > 0)
    def _prologue():
        _fire_idx(0)
        _wait_idx()
        _unpack(0)
        _fire_gather(0)

        @pl.when(nch > 1)
        def _p2():
            _fire_idx(1)

    def _chunk(j, carry):
        @pl.when(j + 1 < nch)
        def _nxt():
            _wait_idx()
            _unpack(j + 1)
            _fire_gather(j + 1)

        _wait_gather()
        pg = _par(j)

        def _grp(g, c2):
            lvec = lbuf[pl.ds(pg * GS + g * 16, 16)]
            abs_ = [pl.multiple_of(lvec[jj] * F, F) for jj in range(16)]
            for jj in range(16):
                ab = abs_[jj]
                e = pg * GS + g * 16 + jj
                for k in range(F // 16):
                    acc[pl.ds(ab + k * 16, 16)] = (
                        acc[pl.ds(ab + k * 16, 16)]
                        + rows[e, pl.ds(k * 16, 16)])
            return c2

        lax.fori_loop(0, GS // 16, _grp, 0)

        @pl.when(j + 2 < nch)
        def _pref():
            _fire_idx(j + 2)

        return carry

    lax.fori_loop(0, nch, _chunk, 0)

    outsz = BW * F
    lastsz = (N - (NT - 1) * BW) * F

    @pl.when(wid < NT - 1)
    def _co():
        pltpu.sync_copy(acc.at[pl.ds(0, outsz)],
                        out_hbm.at[pl.ds(wid * outsz, outsz)])

    @pl.when(wid == NT - 1)
    def _co_last():
        pltpu.sync_copy(acc.at[pl.ds(0, lastsz)],
                        out_hbm.at[pl.ds((NT - 1) * outsz, lastsz)])


def _spmm_body(xa_hbm, pks_a, meta_a, xb_hbm, pks_b, meta_b,
               out_a, out_b,
               metav, pkbuf, gidx, lbuf, rows, acc, semi, semg):
    c = lax.axis_index("c")
    s = lax.axis_index("s")
    wid = s * NC + c
    _spmm_one(xa_hbm, pks_a, meta_a, out_a,
              metav, pkbuf, gidx, lbuf, rows, acc, semi, semg, wid)
    _spmm_one(xb_hbm, pks_b, meta_b, out_b,
              metav, pkbuf, gidx, lbuf, rows, acc, semi, semg, wid)


@functools.cache
def _spmm_kernel():
    return pl.kernel(
        _spmm_body,
        out_type=(jax.ShapeDtypeStruct((N * F,), f32),
                  jax.ShapeDtypeStruct((N * F,), f32)),
        mesh=_sc_mesh(),
        scratch_types=[
            pltpu.VMEM((384,), i32),
            pltpu.VMEM((2 * GS,), i32),
            pltpu.VMEM((2 * GS,), i32),
            pltpu.VMEM((2 * GS,), i32),
            pltpu.VMEM((2 * GS, F), f32),
            pltpu.VMEM((ACC_R * F,), f32),
            pltpu.SemaphoreType.DMA,
            pltpu.SemaphoreType.DMA,
        ],
        name="sc_spmm",
    )


BLK = 1000


def _leaky(x):
    return jnp.where(x >= 0, x, ALPHA * x)


def _k1_body(u, v, w1, w2, wu2, wi2, bu, bi, s1, s2, pu, pv):
    uf = u[...]
    vf = v[...]
    s1[...] = jnp.dot(uf, w1[...], preferred_element_type=f32)
    s2[...] = jnp.dot(vf, w2[...], preferred_element_type=f32)
    pu[...] = jnp.dot(uf, wu2[...], preferred_element_type=f32) + bu[...]
    pv[...] = jnp.dot(vf, wi2[...], preferred_element_type=f32) + bi[...]


def _k3_body(t1, b1, w3, t2, b2, w4, s3, s4):
    h1 = _leaky(t1[...] + b1[...])
    h2 = _leaky(t2[...] + b2[...])
    s3[...] = jnp.dot(h1, w3[...], preferred_element_type=f32)
    s4[...] = jnp.dot(h2, w4[...], preferred_element_type=f32)


def _k5_body(t3, b3, wu1, pu, t4, b4, wi1, pv, user, item):
    h3 = _leaky(t3[...] + b3[...])
    h4 = _leaky(t4[...] + b4[...])
    user[...] = jnp.dot(h3, wu1[...], preferred_element_type=f32) + pu[...]
    item[...] = jnp.dot(h4, wi1[...], preferred_element_type=f32) + pv[...]


_x_spec = pl.BlockSpec((BLK, F), lambda i: (i, 0))
_w_spec = pl.BlockSpec((F, F), lambda i: (0, 0))
_b_spec = pl.BlockSpec((1, F), lambda i: (0, 0))
_o_sd = jax.ShapeDtypeStruct((N, F), f32)

_k1 = pl.pallas_call(
    _k1_body,
    grid=(N // BLK,),
    in_specs=[_x_spec, _x_spec, _w_spec, _w_spec, _w_spec, _w_spec,
              _b_spec, _b_spec],
    out_specs=[_x_spec, _x_spec, _x_spec, _x_spec],
    out_shape=[_o_sd, _o_sd, _o_sd, _o_sd],
)

_k3 = pl.pallas_call(
    _k3_body,
    grid=(N // BLK,),
    in_specs=[_x_spec, _b_spec, _w_spec, _x_spec, _b_spec, _w_spec],
    out_specs=[_x_spec, _x_spec],
    out_shape=[_o_sd, _o_sd],
)

_k5 = pl.pallas_call(
    _k5_body,
    grid=(N // BLK,),
    in_specs=[_x_spec, _b_spec, _w_spec, _x_spec,
              _x_spec, _b_spec, _w_spec, _x_spec],
    out_specs=[_x_spec, _x_spec],
    out_shape=[_o_sd, _o_sd],
)


def _rank(dst, src):
    d3 = dst.reshape(NBLK, BE, 1)
    h3 = _hist(d3)
    sb3, meta8 = _pref(h3)
    pos3, pk3 = _pos(d3, src.reshape(NBLK, BE, 1), sb3)
    return pos3.reshape(E), pk3.reshape(E), meta8.reshape(1024)


def kernel(ufea, vfea, UV_adj, VU_adj, gc1_W, gc1_b, gc2_W, gc2_b,
           gc3m_W, gc3m_b, gc3s_W, gc3s_b, gc4m_W, gc4m_b, gc4s_W, gc4s_b,
           uum_W, uum_b, uus_W, uus_b, ium_W, ium_b, ius_W, ius_b):
    u_idx, i_idx = UV_adj[0], UV_adj[1]
    pos_u, pk_u, meta_u = _rank(u_idx, i_idx)
    pos_i, pk_i, meta_i = _rank(i_idx, u_idx)
    pks_u, pks_i = _perm_kernel()(pos_u, pk_u, pos_i, pk_i)

    def _unchunk(o):
        return o.reshape(N, F)

    s1, s2, pu, pv = _k1(ufea, vfea, gc1_W, gc2_W, uum_W[F:], ium_W[F:],
                         uum_b.reshape(1, F), ium_b.reshape(1, F))
    spmm = _spmm_kernel()
    t1c, t2c = spmm(s1, pks_i, meta_i, s2, pks_u, meta_u)
    t1, t2 = _unchunk(t1c), _unchunk(t2c)
    s3, s4 = _k3(t1, gc1_b.reshape(1, F), gc3m_W,
                 t2, gc2_b.reshape(1, F), gc4m_W)
    t3c, t4c = spmm(s3, pks_u, meta_u, s4, pks_i, meta_i)
    t3, t4 = _unchunk(t3c), _unchunk(t4c)
    user, item = _k5(t3, gc3m_b.reshape(1, F), uum_W[:F], pu,
                     t4, gc4m_b.reshape(1, F), ium_W[:F], pv)
    return (user, item)
